# baseline (speedup 1.0000x reference)
import functools

import jax
import jax.numpy as jnp
import numpy as np
from jax import lax
from jax.experimental import pallas as pl
from jax.experimental.pallas import tpu as pltpu
from jax.experimental.pallas import tpu_sc as plsc

N = 10000
K = 16
SIZE = 128
HEADS = 8
ATT = 32
HA = HEADS * ATT
DIST_KERNELS = 16
MAX_DIST = 20.0
SEQ_SIZE = 20

T = 256
NP = 10240
GRID = NP // T
E = T * K
EP = NP * K
CH = 128
NW = 32
SCALE = 1.0 / np.sqrt(ATT)


def _np_seg():
    seg = np.zeros((HA, HEADS), np.float32)
    for h in range(HEADS):
        seg[h * ATT:(h + 1) * ATT, h] = 1.0
    return seg

_SEG = _np_seg()
_SEGT = _SEG.T.copy()

_C3 = np.zeros((16, 1), np.float32)
_C3[0:3, 0] = 1.0

_MD = np.zeros((16, 9), np.float32)
for _i in range(3):
    for _j in range(3):
        _MD[_j, 3 * _i + _j] = 1.0
_MO = np.zeros((16, 9), np.float32)
for _i in range(3):
    for _j in range(3):
        _MO[3 + 3 * _i + _j, 3 * _i + _j] = 1.0
_MG = np.zeros((9, 3), np.float32)
for _i in range(3):
    for _j in range(3):
        _MG[3 * _i + _j, _i] = 1.0
_MU2 = np.zeros((16, 27), np.float32)
_MV = np.zeros((16, 27), np.float32)
_MS = np.zeros((27, 9), np.float32)
for _jj in range(3):
    for _ii in range(3):
        for _ll in range(3):
            _c = 9 * _jj + 3 * _ii + _ll
            _MU2[3 + 3 * _jj + _ii, _c] = 1.0
            _MV[3 + 3 * _jj + _ll, _c] = 1.0
            _MS[_c, 3 * _ii + _ll] = 1.0
_MQ = np.zeros((9, 4), np.float32)
for _i, _signs in enumerate([(1, 1, 1), (1, -1, -1), (-1, 1, -1), (-1, -1, 1)]):
    _MQ[0, _i], _MQ[4, _i], _MQ[8, _i] = _signs
_MSGN = np.zeros((9, 4), np.float32)
_MSGN[7, 1], _MSGN[5, 1] = 1.0, -1.0
_MSGN[2, 2], _MSGN[6, 2] = 1.0, -1.0
_MSGN[3, 3], _MSGN[1, 3] = 1.0, -1.0
_CEN = np.linspace(0.0, MAX_DIST, DIST_KERNELS).astype(np.float32)[None, :]
_INV_SIG = DIST_KERNELS / MAX_DIST
_MROT = np.eye(16, dtype=np.float32)
_MROT[14, 14] = _MROT[15, 15] = 0.0
_MROT[14, 15] = _MROT[15, 14] = 1.0
_GSIN = np.zeros((16, 2), np.float32)
_GSIN[14, 0], _GSIN[15, 0] = 1.0, -1.0
_GCOS = np.zeros((16, 2), np.float32)
_GCOS[14, 1] = _GCOS[15, 1] = 1.0
_GMISC = np.zeros((32, 2), np.float32)
_GMISC[13, 0] = 1.0
_GMISC[16, 1] = 1.0
_O0 = np.array([[1.0, 0.0, 0.0, 0.0]], np.float32)


def _sc_gather(table, idx2d, d):
    rows = idx2d.shape[0] * CH
    nch = rows // CH // NW
    dt = table.dtype
    itemsize = 2 if dt == jnp.bfloat16 else 4
    ntab = table.shape[0]
    idx_bytes = nch * CH * 4
    buf_bytes = CH * d * itemsize
    per_sub = (7_600_000 - ntab * d * itemsize) // 16
    NBUF = max(2, min(4, (per_sub - idx_bytes) // buf_bytes))
    assert nch % NBUF == 0

    mesh = plsc.VectorSubcoreMesh(core_axis_name="c", subcore_axis_name="s",
                                  num_cores=2, num_subcores=16)

    @functools.partial(
        pl.kernel,
        out_type=jax.ShapeDtypeStruct((rows, d), dt),
        mesh=mesh,
        compiler_params=pltpu.CompilerParams(use_tc_tiling_on_sc=True),
        scratch_types=[
            pltpu.VMEM((nch, CH), jnp.int32),
            pltpu.VMEM_SHARED((ntab, d), dt),
        ] + [pltpu.VMEM((CH, d), dt)] * NBUF
          + [pltpu.SemaphoreType.DMA] * (2 * NBUF),
    )
    def gk(table_hbm, idx_hbm, out_hbm, idx_v, tab_s, *bufsem):
        bufs = bufsem[:NBUF]
        gsem = bufsem[NBUF:2 * NBUF]
        wsem = bufsem[2 * NBUF:]
        sid = lax.axis_index("s")
        wid = sid * 2 + lax.axis_index("c")
        row0 = wid * nch

        @pl.when(sid == 0)
        def _():
            pltpu.sync_copy(table_hbm, tab_s)

        pltpu.sync_copy(idx_hbm.at[pl.ds(row0, nch)], idx_v)
        plsc.subcore_barrier()

        for b in range(NBUF):
            pltpu.async_copy(tab_s.at[idx_v.at[b]], bufs[b], gsem[b])

        def outer(g, _):
            base = g * NBUF
            for b in range(NBUF):
                i = base + b
                pltpu.make_async_copy(tab_s.at[idx_v.at[i]],
                                      bufs[b], gsem[b]).wait()
                pltpu.async_copy(bufs[b],
                                 out_hbm.at[pl.ds((row0 + i) * CH, CH)], wsem[b])
            for b in range(NBUF):
                nxt = base + NBUF + b

                @pl.when(nxt < nch)
                def _():
                    pltpu.make_async_copy(
                        bufs[b], out_hbm.at[pl.ds((row0 + base + b) * CH, CH)],
                        wsem[b]).wait()
                    pltpu.async_copy(tab_s.at[idx_v.at[nxt]], bufs[b], gsem[b])
            return 0

        lax.fori_loop(0, nch // NBUF, outer, 0)
        for b in range(NBUF):
            i = nch - NBUF + b
            pltpu.make_async_copy(bufs[b],
                                  out_hbm.at[pl.ds((row0 + i) * CH, CH)],
                                  wsem[b]).wait()

    return gk(table, idx2d)


def _rel_body(dist_ref, distg_ref, c3, md, mo, mg, mu2, mv, ms, mq, msgn, cen,
              mrot, gsin, gcos, o0, rel_ref):
    f32 = jnp.float32
    s0 = dist_ref[:, :16]
    y32 = distg_ref[:, :32]
    y = y32[:, :16]
    s = jnp.broadcast_to(s0[:, None, :], (T, K, 16)).reshape(E, 16)
    dd = y - s
    d2 = jnp.dot(dd * dd, c3[...], preferred_element_type=f32)
    dist = jnp.sqrt(d2 + 1e-12)
    vb = jnp.dot(dd, md[...], preferred_element_type=f32)
    ob = jnp.dot(s, mo[...], preferred_element_type=f32)
    rcp = 1.0 / (dist + 1e-6)
    direction = jnp.dot(ob * vb, mg[...], preferred_element_type=f32) * rcp
    u = jnp.dot(s, mu2[...], preferred_element_type=f32)
    v = jnp.dot(y, mv[...], preferred_element_type=f32)
    r9 = jnp.dot(u * v, ms[...], preferred_element_type=f32)
    args = jnp.maximum(1.0 + jnp.dot(r9, mq[...],
                                     preferred_element_type=f32), 1e-6)
    w4 = 0.5 * jnp.sqrt(args)
    sgn = jnp.sign(jnp.dot(r9, msgn[...], preferred_element_type=f32))
    quat = w4 * (sgn + o0[...])
    yrot = jnp.dot(y, mrot[...], preferred_element_type=f32)
    p = s * yrot
    q = s * y
    dsc = (jnp.dot(p, gsin[...], preferred_element_type=f32)
           + jnp.dot(q, gcos[...], preferred_element_type=f32))
    z = (dist - cen[...]) * _INV_SIG
    rbf = jnp.exp(-(z * z))
    rel_ref[...] = jnp.concatenate(
        [rbf, direction, quat, dsc, y32[:, 13:14], y32[:, 16:17],
         jnp.zeros((E, 5), f32)], axis=1)


def _rel_call(dist16p, distg):
    consts = [jnp.asarray(a) for a in
              (_C3, _MD, _MO, _MG, _MU2, _MV, _MS, _MQ, _MSGN, _CEN,
               _MROT, _GSIN, _GCOS, _O0)]
    return pl.pallas_call(
        _rel_body,
        grid=(GRID,),
        in_specs=[pl.BlockSpec((T, SIZE), lambda i: (i, 0)),
                  pl.BlockSpec((E, SIZE), lambda i: (i, 0))]
                 + [_wspec(a.shape) for a in consts],
        out_specs=pl.BlockSpec((E, 32), lambda i: (i, 0)),
        out_shape=jax.ShapeDtypeStruct((EP, 32), jnp.float32),
    )(dist16p, distg, *consts)


def _a1_body(f_ref, d_ref, wpre, bpre, w0, b0, w1, b1, out0_ref, loc_ref,
             d16_ref):
    f32 = jnp.float32
    x = f_ref[...]
    out0 = jnp.dot(x, wpre[...], preferred_element_type=f32) + bpre[...]
    h = jnp.maximum(out0, 0.0)
    h = jnp.maximum(jnp.dot(h, w0[...], preferred_element_type=f32) + b0[...], 0.0)
    loc = jnp.maximum(jnp.dot(h, w1[...], preferred_element_type=f32) + b1[...], 0.0)
    out0_ref[...] = out0
    loc_ref[...] = loc
    d = d_ref[...]
    ang = d[:, 12:13]
    i = pl.program_id(0)
    nid = (i * T + lax.broadcasted_iota(jnp.int32, (T, 1), 0)).astype(f32)
    d16_ref[...] = jnp.concatenate(
        [d[:, :14], jnp.sin(ang), jnp.cos(ang), nid, jnp.zeros((T, 111), f32)],
        axis=1)


def _wspec(shape):
    return pl.BlockSpec(shape, lambda i: tuple(0 for _ in shape))


def _a1_call(fp, dist16p, wpre, bpre, w0, b0, w1, b1):
    return pl.pallas_call(
        _a1_body,
        grid=(GRID,),
        in_specs=[pl.BlockSpec((T, SIZE), lambda i: (i, 0)),
                  pl.BlockSpec((T, 16), lambda i: (i, 0)),
                  _wspec((SIZE, SIZE)), _wspec((1, SIZE)),
                  _wspec((SIZE, SIZE)), _wspec((1, SIZE)),
                  _wspec((SIZE, SIZE)), _wspec((1, SIZE))],
        out_specs=[pl.BlockSpec((T, SIZE), lambda i: (i, 0)),
                   pl.BlockSpec((T, SIZE), lambda i: (i, 0)),
                   pl.BlockSpec((T, SIZE), lambda i: (i, 0))],
        out_shape=[jax.ShapeDtypeStruct((NP, SIZE), jnp.float32),
                   jax.ShapeDtypeStruct((NP, SIZE), jnp.float32),
                   jax.ShapeDtypeStruct((NP, SIZE), jnp.float32)],
    )(fp, dist16p, wpre, bpre, w0, b0, w1, b1)


def _attention_core(q, k, v, seg, segt):
    qe = jnp.broadcast_to(q[:, None, :], (T, K, HA)).reshape(E, HA)
    logits = jnp.dot(qe * k, seg[...],
                     preferred_element_type=jnp.float32) * SCALE
    l3 = logits.reshape(T, K, HEADS)
    m = jnp.max(l3, axis=1, keepdims=True)
    e3 = jnp.exp(l3 - m)
    ssum = jnp.sum(e3, axis=1, keepdims=True)
    w3 = e3 / ssum
    we = jnp.dot(w3.reshape(E, HEADS), segt[...],
                 preferred_element_type=jnp.float32)
    return jnp.sum((we * v).reshape(T, K, HA), axis=1)


def _next_local(out, nw0, nb0, nw1, nb1):
    f32 = jnp.float32
    h = jnp.maximum(out, 0.0)
    h = jnp.maximum(jnp.dot(h, nw0[...], preferred_element_type=f32) + nb0[...], 0.0)
    return jnp.maximum(jnp.dot(h, nw1[...], preferred_element_type=f32) + nb1[...], 0.0)


def _benc_body(combine_out, feats_ref, loc_ref, lg_ref, rel_ref,
               wq, bq, wkf, wkr, bk, wvf, wvr, bv, wo, bo,
               nw0, nb0, nw1, nb1, seg, segt, out_ref):
    f32 = jnp.float32
    loc = loc_ref[...]
    q = jnp.dot(loc, wq[...], preferred_element_type=f32) + bq[...]
    lg = lg_ref[...]
    rel = rel_ref[...]
    k = (jnp.dot(lg, wkf[...], preferred_element_type=f32)
         + jnp.dot(rel, wkr[...], preferred_element_type=f32) + bk[...])
    v = (jnp.dot(lg, wvf[...], preferred_element_type=f32)
         + jnp.dot(rel, wvr[...], preferred_element_type=f32) + bv[...])
    o = _attention_core(q, k, v, seg, segt)
    att = jnp.dot(o, wo[...], preferred_element_type=f32) + bo[...]
    out = feats_ref[...] + att
    nloc = _next_local(out, nw0, nb0, nw1, nb1)
    out_ref[0][...] = out
    out_ref[1][...] = nloc


def _benc_call(combine_out, feats, loc, lg, relp, wq, bq, wkf, wkr, bk,
               wvf, wvr, bv, wo, bo, nw0, nb0, nw1, nb1):
    out_specs = [pl.BlockSpec((T, SIZE), lambda i: (i, 0))] * 2
    out_shape = [jax.ShapeDtypeStruct((NP, SIZE), jnp.float32)] * 2

    def body(*refs):
        _benc_body(combine_out, *refs[:20], refs[20:])

    return pl.pallas_call(
        body,
        grid=(GRID,),
        in_specs=[pl.BlockSpec((T, SIZE), lambda i: (i, 0)),
                  pl.BlockSpec((T, SIZE), lambda i: (i, 0)),
                  pl.BlockSpec((E, SIZE), lambda i: (i, 0)),
                  pl.BlockSpec((E, 32), lambda i: (i, 0)),
                  _wspec((SIZE, HA)), _wspec((1, HA)),
                  _wspec((SIZE, HA)), _wspec((32, HA)), _wspec((1, HA)),
                  _wspec((SIZE, HA)), _wspec((32, HA)), _wspec((1, HA)),
                  _wspec((HA, SIZE)), _wspec((1, SIZE)),
                  _wspec((SIZE, SIZE)), _wspec((1, SIZE)),
                  _wspec((SIZE, SIZE)), _wspec((1, SIZE)),
                  _wspec((HA, HEADS)), _wspec((HEADS, HA))],
        out_specs=out_specs,
        out_shape=out_shape,
    )(feats, loc, lg, relp, wq, bq, wkf, wkr, bk, wvf, wvr, bv, wo, bo,
      nw0, nb0, nw1, nb1, jnp.asarray(_SEG), jnp.asarray(_SEGT))


def _bdec_body(final, feats_ref, loc_ref, lgd_ref, enc_ref, rel_ref,
               wq, bq, wkf, wkr, wks, bk, wvf, wvr, wvs, bv, wo, bo,
               nw0, nb0, nw1, nb1, seg, segt, out_ref=None):
    f32 = jnp.float32
    i = pl.program_id(0)
    rel = rel_ref[...]
    eidx = lax.broadcasted_iota(jnp.int32, (E, 1), 0)
    nid = (i * T + eidx // K).astype(f32)
    pre = rel[:, 26:27] < nid
    lgd = lgd_ref[...]
    encg = enc_ref[...]
    feat_part = jnp.where(pre, lgd, encg)
    seqg = rel[:, 25:26].astype(jnp.int32)
    io32 = lax.broadcasted_iota(jnp.int32, (E, 32), 1)
    oh = jnp.where((seqg == io32) & pre, 1.0, 0.0)
    loc = loc_ref[...]
    q = jnp.dot(loc, wq[...], preferred_element_type=f32) + bq[...]
    k = (jnp.dot(feat_part, wkf[...], preferred_element_type=f32)
         + jnp.dot(rel, wkr[...], preferred_element_type=f32)
         + jnp.dot(oh, wks[...], preferred_element_type=f32) + bk[...])
    v = (jnp.dot(feat_part, wvf[...], preferred_element_type=f32)
         + jnp.dot(rel, wvr[...], preferred_element_type=f32)
         + jnp.dot(oh, wvs[...], preferred_element_type=f32) + bv[...])
    o = _attention_core(q, k, v, seg, segt)
    att = jnp.dot(o, wo[...], preferred_element_type=f32) + bo[...]
    out = feats_ref[...] + att
    if final:
        wpost, bpost = nw0, nb0
        out_ref[...] = jnp.dot(out, wpost[...], preferred_element_type=f32) + bpost[...]
    else:
        nloc = _next_local(out, nw0, nb0, nw1, nb1)
        out_ref[0][...] = out
        out_ref[1][...] = nloc


def _bdec_call(final, feats, loc, lgd, encg, relp,
               wq, bq, wkf, wkr, wks, bk, wvf, wvr, wvs, bv, wo, bo,
               nw0, nb0, nw1, nb1, feats_spec, loc_spec, lgd_spec, enc_spec):
    if final:
        out_specs = pl.BlockSpec((T, 32), lambda i: (i, 0))
        out_shape = jax.ShapeDtypeStruct((NP, 32), jnp.float32)
        mlp_specs = [_wspec((SIZE, 32)), _wspec((1, 32))]
        mlp_args = (nw0, nb0)
    else:
        out_specs = [pl.BlockSpec((T, SIZE), lambda i: (i, 0))] * 2
        out_shape = [jax.ShapeDtypeStruct((NP, SIZE), jnp.float32)] * 2
        mlp_specs = [_wspec((SIZE, SIZE)), _wspec((1, SIZE)),
                     _wspec((SIZE, SIZE)), _wspec((1, SIZE))]
        mlp_args = (nw0, nb0, nw1, nb1)

    def body(*refs):
        if final:
            ins = list(refs[:19]) + [None, None] + list(refs[19:21])
            _bdec_body(final, *ins, refs[21])
        else:
            _bdec_body(final, *refs[:23], refs[23:])

    return pl.pallas_call(
        body,
        grid=(GRID,),
        in_specs=[feats_spec, loc_spec, lgd_spec, enc_spec,
                  pl.BlockSpec((E, 32), lambda i: (i, 0)),
                  _wspec((SIZE, HA)), _wspec((1, HA)),
                  _wspec((SIZE, HA)), _wspec((32, HA)), _wspec((32, HA)),
                  _wspec((1, HA)),
                  _wspec((SIZE, HA)), _wspec((32, HA)), _wspec((32, HA)),
                  _wspec((1, HA)),
                  _wspec((HA, SIZE)), _wspec((1, SIZE))] + mlp_specs
                 + [_wspec((HA, HEADS)), _wspec((HEADS, HA))],
        out_specs=out_specs,
        out_shape=out_shape,
    )(feats, loc, lgd, encg, relp, wq, bq, wkf, wkr, wks, bk,
      wvf, wvr, wvs, bv, wo, bo, *mlp_args,
      jnp.asarray(_SEG), jnp.asarray(_SEGT))


def _row(b):
    return b.reshape(1, -1)


def _pad_rows(w, rows):
    return jnp.pad(w, ((0, rows - w.shape[0]), (0, 0)))


def _enc_weights(bp):
    wk = bp["Wk"]["W"]
    wv = bp["Wv"]["W"]
    return dict(
        wq=bp["Wq"]["W"], bq=_row(bp["Wq"]["b"]),
        wkf=wk[:SIZE], wkr=_pad_rows(wk[SIZE:SIZE + 25], 32),
        bk=_row(bp["Wk"]["b"]),
        wvf=wv[:SIZE], wvr=_pad_rows(wv[SIZE:SIZE + 25], 32),
        bv=_row(bp["Wv"]["b"]),
        wo=bp["Wo"]["W"], bo=_row(bp["Wo"]["b"]),
    )


def _dec_weights(bp):
    wk = bp["Wk"]["W"]
    wv = bp["Wv"]["W"]
    d = _enc_weights(bp)
    d["wks"] = _pad_rows(wk[SIZE + 25:SIZE + 25 + SEQ_SIZE], 32)
    d["wvs"] = _pad_rows(wv[SIZE + 25:SIZE + 25 + SEQ_SIZE], 32)
    return d


def _mlp_weights(bp):
    return (bp["mlp0"]["W"], _row(bp["mlp0"]["b"]),
            bp["mlp1"]["W"], _row(bp["mlp1"]["b"]))


def kernel(features, sequence, distances, structure, params):
    f32 = jnp.float32
    fp = jnp.pad(features.astype(f32), ((0, NP - N), (0, 0)))
    dist16 = jnp.concatenate(
        [distances.astype(f32), sequence.astype(f32)[:, None],
         jnp.zeros((N, 2), f32)], axis=1)
    dist16p = jnp.pad(dist16, ((0, NP - N), (0, 0)))
    connp = jnp.pad(structure.astype(jnp.int32), ((0, NP - N), (0, 0)))
    idx2d = connp.reshape(EP // CH, CH)

    eb0, eb1 = params["enc_blocks"]
    db0, db1 = params["dec_blocks"]
    ew0, ew1 = _enc_weights(eb0), _enc_weights(eb1)
    dw0, dw1 = _dec_weights(db0), _dec_weights(db1)
    wpost = _pad_rows(params["post"]["W"].T, 32).T
    bpost = _pad_rows(_row(params["post"]["b"]).T, 32).T

    out0, loc1, dist16f = _a1_call(fp, dist16p, params["enc_pre"]["W"],
                                   _row(params["enc_pre"]["b"]),
                                   *_mlp_weights(eb0))

    distg = _sc_gather(dist16f, idx2d, SIZE)
    relp = _rel_call(dist16f, distg)

    lg1 = _sc_gather(loc1, idx2d, SIZE)
    out1, loc2 = _benc_call(False, out0, loc1, lg1, relp,
                            **{k: v for k, v in ew0.items()},
                            nw0=_mlp_weights(eb1)[0], nb0=_mlp_weights(eb1)[1],
                            nw1=_mlp_weights(eb1)[2], nb1=_mlp_weights(eb1)[3])

    lg2 = _sc_gather(loc2, idx2d, SIZE)
    out2, loc3 = _benc_call(False, out1, loc2, lg2, relp,
                            **{k: v for k, v in ew1.items()},
                            nw0=_mlp_weights(db0)[0], nb0=_mlp_weights(db0)[1],
                            nw1=_mlp_weights(db0)[2], nb1=_mlp_weights(db0)[3])

    lgE = _sc_gather(out2, idx2d, SIZE)
    lg3 = _sc_gather(loc3, idx2d, SIZE)

    tspec0 = pl.BlockSpec((T, SIZE), lambda i: (i, 0))
    espec0 = pl.BlockSpec((E, SIZE), lambda i: (i, 0))

    out3, loc4 = _bdec_call(False, out2, loc3, lg3, lgE, relp,
                            dw0["wq"], dw0["bq"], dw0["wkf"], dw0["wkr"],
                            dw0["wks"], dw0["bk"], dw0["wvf"], dw0["wvr"],
                            dw0["wvs"], dw0["bv"], dw0["wo"], dw0["bo"],
                            _mlp_weights(db1)[0], _mlp_weights(db1)[1],
                            _mlp_weights(db1)[2], _mlp_weights(db1)[3],
                            tspec0, tspec0, espec0, espec0)

    lg4 = _sc_gather(loc4, idx2d, SIZE)
    final32 = _bdec_call(True, out3, loc4, lg4, lgE, relp,
                         dw1["wq"], dw1["bq"], dw1["wkf"], dw1["wkr"],
                         dw1["wks"], dw1["bk"], dw1["wvf"], dw1["wvr"],
                         dw1["wvs"], dw1["bv"], dw1["wo"], dw1["bo"],
                         wpost, bpost, wpost, bpost,
                         tspec0, tspec0, espec0, espec0)

    return final32[:N, :20]


# speedup vs baseline: 1.0066x; 1.0066x over previous
import functools

import jax
import jax.numpy as jnp
import numpy as np
from jax import lax
from jax.experimental import pallas as pl
from jax.experimental.pallas import tpu as pltpu
from jax.experimental.pallas import tpu_sc as plsc

N = 10000
K = 16
SIZE = 128
HEADS = 8
ATT = 32
HA = HEADS * ATT
DIST_KERNELS = 16
MAX_DIST = 20.0
SEQ_SIZE = 20

T = 320
NP = 10240
GRID = NP // T
E = T * K
EP = NP * K
CH = 128
NW = 32
SCALE = 1.0 / np.sqrt(ATT)


def _np_seg():
    seg = np.zeros((HA, HEADS), np.float32)
    for h in range(HEADS):
        seg[h * ATT:(h + 1) * ATT, h] = 1.0
    return seg

_SEG = _np_seg()
_SEGT = _SEG.T.copy()

_C3 = np.zeros((16, 1), np.float32)
_C3[0:3, 0] = 1.0

_MD = np.zeros((16, 9), np.float32)
for _i in range(3):
    for _j in range(3):
        _MD[_j, 3 * _i + _j] = 1.0
_MO = np.zeros((16, 9), np.float32)
for _i in range(3):
    for _j in range(3):
        _MO[3 + 3 * _i + _j, 3 * _i + _j] = 1.0
_MG = np.zeros((9, 3), np.float32)
for _i in range(3):
    for _j in range(3):
        _MG[3 * _i + _j, _i] = 1.0
_MU2 = np.zeros((16, 27), np.float32)
_MV = np.zeros((16, 27), np.float32)
_MS = np.zeros((27, 9), np.float32)
for _jj in range(3):
    for _ii in range(3):
        for _ll in range(3):
            _c = 9 * _jj + 3 * _ii + _ll
            _MU2[3 + 3 * _jj + _ii, _c] = 1.0
            _MV[3 + 3 * _jj + _ll, _c] = 1.0
            _MS[_c, 3 * _ii + _ll] = 1.0
_MQ = np.zeros((9, 4), np.float32)
for _i, _signs in enumerate([(1, 1, 1), (1, -1, -1), (-1, 1, -1), (-1, -1, 1)]):
    _MQ[0, _i], _MQ[4, _i], _MQ[8, _i] = _signs
_MSGN = np.zeros((9, 4), np.float32)
_MSGN[7, 1], _MSGN[5, 1] = 1.0, -1.0
_MSGN[2, 2], _MSGN[6, 2] = 1.0, -1.0
_MSGN[3, 3], _MSGN[1, 3] = 1.0, -1.0
_CEN = np.linspace(0.0, MAX_DIST, DIST_KERNELS).astype(np.float32)[None, :]
_INV_SIG = DIST_KERNELS / MAX_DIST
_MROT = np.eye(16, dtype=np.float32)
_MROT[14, 14] = _MROT[15, 15] = 0.0
_MROT[14, 15] = _MROT[15, 14] = 1.0
_GSIN = np.zeros((16, 2), np.float32)
_GSIN[14, 0], _GSIN[15, 0] = 1.0, -1.0
_GCOS = np.zeros((16, 2), np.float32)
_GCOS[14, 1] = _GCOS[15, 1] = 1.0
_GMISC = np.zeros((32, 2), np.float32)
_GMISC[13, 0] = 1.0
_GMISC[16, 1] = 1.0
_O0 = np.array([[1.0, 0.0, 0.0, 0.0]], np.float32)


def _sc_gather(table, idx2d, d):
    rows = idx2d.shape[0] * CH
    nch = rows // CH // NW
    dt = table.dtype
    itemsize = 2 if dt == jnp.bfloat16 else 4
    ntab = table.shape[0]
    idx_bytes = nch * CH * 4
    buf_bytes = CH * d * itemsize
    per_sub = (7_600_000 - ntab * d * itemsize) // 16
    NBUF = max(2, min(4, (per_sub - idx_bytes) // buf_bytes))
    assert nch % NBUF == 0

    mesh = plsc.VectorSubcoreMesh(core_axis_name="c", subcore_axis_name="s",
                                  num_cores=2, num_subcores=16)

    @functools.partial(
        pl.kernel,
        out_type=jax.ShapeDtypeStruct((rows, d), dt),
        mesh=mesh,
        compiler_params=pltpu.CompilerParams(use_tc_tiling_on_sc=True),
        scratch_types=[
            pltpu.VMEM((nch, CH), jnp.int32),
            pltpu.VMEM_SHARED((ntab, d), dt),
        ] + [pltpu.VMEM((CH, d), dt)] * NBUF
          + [pltpu.SemaphoreType.DMA] * (2 * NBUF),
    )
    def gk(table_hbm, idx_hbm, out_hbm, idx_v, tab_s, *bufsem):
        bufs = bufsem[:NBUF]
        gsem = bufsem[NBUF:2 * NBUF]
        wsem = bufsem[2 * NBUF:]
        sid = lax.axis_index("s")
        wid = sid * 2 + lax.axis_index("c")
        row0 = wid * nch

        @pl.when(sid == 0)
        def _():
            pltpu.sync_copy(table_hbm, tab_s)

        pltpu.sync_copy(idx_hbm.at[pl.ds(row0, nch)], idx_v)
        plsc.subcore_barrier()

        for b in range(NBUF):
            pltpu.async_copy(tab_s.at[idx_v.at[b]], bufs[b], gsem[b])

        def outer(g, _):
            base = g * NBUF
            for b in range(NBUF):
                i = base + b
                pltpu.make_async_copy(tab_s.at[idx_v.at[i]],
                                      bufs[b], gsem[b]).wait()
                pltpu.async_copy(bufs[b],
                                 out_hbm.at[pl.ds((row0 + i) * CH, CH)], wsem[b])
            for b in range(NBUF):
                nxt = base + NBUF + b

                @pl.when(nxt < nch)
                def _():
                    pltpu.make_async_copy(
                        bufs[b], out_hbm.at[pl.ds((row0 + base + b) * CH, CH)],
                        wsem[b]).wait()
                    pltpu.async_copy(tab_s.at[idx_v.at[nxt]], bufs[b], gsem[b])
            return 0

        lax.fori_loop(0, nch // NBUF, outer, 0)
        for b in range(NBUF):
            i = nch - NBUF + b
            pltpu.make_async_copy(bufs[b],
                                  out_hbm.at[pl.ds((row0 + i) * CH, CH)],
                                  wsem[b]).wait()

    return gk(table, idx2d)


def _rel_body(dist_ref, distg_ref, c3, md, mo, mg, mu2, mv, ms, mq, msgn, cen,
              mrot, gsin, gcos, o0, rel_ref):
    f32 = jnp.float32
    s0 = dist_ref[:, :16]
    y32 = distg_ref[:, :32]
    y = y32[:, :16]
    s = jnp.broadcast_to(s0[:, None, :], (T, K, 16)).reshape(E, 16)
    dd = y - s
    d2 = jnp.dot(dd * dd, c3[...], preferred_element_type=f32)
    dist = jnp.sqrt(d2 + 1e-12)
    vb = jnp.dot(dd, md[...], preferred_element_type=f32)
    ob = jnp.dot(s, mo[...], preferred_element_type=f32)
    rcp = 1.0 / (dist + 1e-6)
    direction = jnp.dot(ob * vb, mg[...], preferred_element_type=f32) * rcp
    u = jnp.dot(s, mu2[...], preferred_element_type=f32)
    v = jnp.dot(y, mv[...], preferred_element_type=f32)
    r9 = jnp.dot(u * v, ms[...], preferred_element_type=f32)
    args = jnp.maximum(1.0 + jnp.dot(r9, mq[...],
                                     preferred_element_type=f32), 1e-6)
    w4 = 0.5 * jnp.sqrt(args)
    sgn = jnp.sign(jnp.dot(r9, msgn[...], preferred_element_type=f32))
    quat = w4 * (sgn + o0[...])
    yrot = jnp.dot(y, mrot[...], preferred_element_type=f32)
    p = s * yrot
    q = s * y
    dsc = (jnp.dot(p, gsin[...], preferred_element_type=f32)
           + jnp.dot(q, gcos[...], preferred_element_type=f32))
    z = (dist - cen[...]) * _INV_SIG
    rbf = jnp.exp(-(z * z))
    rel_ref[...] = jnp.concatenate(
        [rbf, direction, quat, dsc, y32[:, 13:14], y32[:, 16:17],
         jnp.zeros((E, 5), f32)], axis=1)


def _rel_call(dist16p, distg):
    consts = [jnp.asarray(a) for a in
              (_C3, _MD, _MO, _MG, _MU2, _MV, _MS, _MQ, _MSGN, _CEN,
               _MROT, _GSIN, _GCOS, _O0)]
    return pl.pallas_call(
        _rel_body,
        grid=(GRID,),
        in_specs=[pl.BlockSpec((T, SIZE), lambda i: (i, 0)),
                  pl.BlockSpec((E, SIZE), lambda i: (i, 0))]
                 + [_wspec(a.shape) for a in consts],
        out_specs=pl.BlockSpec((E, 32), lambda i: (i, 0)),
        out_shape=jax.ShapeDtypeStruct((EP, 32), jnp.float32),
    )(dist16p, distg, *consts)


def _a1_body(f_ref, d_ref, wpre, bpre, w0, b0, w1, b1, out0_ref, loc_ref,
             d16_ref):
    f32 = jnp.float32
    x = f_ref[...]
    out0 = jnp.dot(x, wpre[...], preferred_element_type=f32) + bpre[...]
    h = jnp.maximum(out0, 0.0)
    h = jnp.maximum(jnp.dot(h, w0[...], preferred_element_type=f32) + b0[...], 0.0)
    loc = jnp.maximum(jnp.dot(h, w1[...], preferred_element_type=f32) + b1[...], 0.0)
    out0_ref[...] = out0
    loc_ref[...] = loc
    d = d_ref[...]
    ang = d[:, 12:13]
    i = pl.program_id(0)
    nid = (i * T + lax.broadcasted_iota(jnp.int32, (T, 1), 0)).astype(f32)
    d16_ref[...] = jnp.concatenate(
        [d[:, :14], jnp.sin(ang), jnp.cos(ang), nid, jnp.zeros((T, 111), f32)],
        axis=1)


def _wspec(shape):
    return pl.BlockSpec(shape, lambda i: tuple(0 for _ in shape))


def _a1_call(fp, dist16p, wpre, bpre, w0, b0, w1, b1):
    return pl.pallas_call(
        _a1_body,
        grid=(GRID,),
        in_specs=[pl.BlockSpec((T, SIZE), lambda i: (i, 0)),
                  pl.BlockSpec((T, 16), lambda i: (i, 0)),
                  _wspec((SIZE, SIZE)), _wspec((1, SIZE)),
                  _wspec((SIZE, SIZE)), _wspec((1, SIZE)),
                  _wspec((SIZE, SIZE)), _wspec((1, SIZE))],
        out_specs=[pl.BlockSpec((T, SIZE), lambda i: (i, 0)),
                   pl.BlockSpec((T, SIZE), lambda i: (i, 0)),
                   pl.BlockSpec((T, SIZE), lambda i: (i, 0))],
        out_shape=[jax.ShapeDtypeStruct((NP, SIZE), jnp.float32),
                   jax.ShapeDtypeStruct((NP, SIZE), jnp.float32),
                   jax.ShapeDtypeStruct((NP, SIZE), jnp.float32)],
    )(fp, dist16p, wpre, bpre, w0, b0, w1, b1)


def _attention_core(q, k, v, seg, segt):
    qe = jnp.broadcast_to(q[:, None, :], (T, K, HA)).reshape(E, HA)
    logits = jnp.dot(qe * k, seg[...],
                     preferred_element_type=jnp.float32) * SCALE
    l3 = logits.reshape(T, K, HEADS)
    m = jnp.max(l3, axis=1, keepdims=True)
    e3 = jnp.exp(l3 - m)
    ssum = jnp.sum(e3, axis=1, keepdims=True)
    w3 = e3 / ssum
    we = jnp.dot(w3.reshape(E, HEADS), segt[...],
                 preferred_element_type=jnp.float32)
    return jnp.sum((we * v).reshape(T, K, HA), axis=1)


def _next_local(out, nw0, nb0, nw1, nb1):
    f32 = jnp.float32
    h = jnp.maximum(out, 0.0)
    h = jnp.maximum(jnp.dot(h, nw0[...], preferred_element_type=f32) + nb0[...], 0.0)
    return jnp.maximum(jnp.dot(h, nw1[...], preferred_element_type=f32) + nb1[...], 0.0)


def _benc_body(combine_out, feats_ref, loc_ref, lg_ref, rel_ref,
               wq, bq, wkf, wkr, bk, wvf, wvr, bv, wo, bo,
               nw0, nb0, nw1, nb1, seg, segt, out_ref):
    f32 = jnp.float32
    loc = loc_ref[...]
    q = jnp.dot(loc, wq[...], preferred_element_type=f32) + bq[...]
    lg = lg_ref[...]
    rel = rel_ref[...]
    k = (jnp.dot(lg, wkf[...], preferred_element_type=f32)
         + jnp.dot(rel, wkr[...], preferred_element_type=f32) + bk[...])
    v = (jnp.dot(lg, wvf[...], preferred_element_type=f32)
         + jnp.dot(rel, wvr[...], preferred_element_type=f32) + bv[...])
    o = _attention_core(q, k, v, seg, segt)
    att = jnp.dot(o, wo[...], preferred_element_type=f32) + bo[...]
    out = feats_ref[...] + att
    nloc = _next_local(out, nw0, nb0, nw1, nb1)
    out_ref[0][...] = out
    out_ref[1][...] = nloc


def _benc_call(combine_out, feats, loc, lg, relp, wq, bq, wkf, wkr, bk,
               wvf, wvr, bv, wo, bo, nw0, nb0, nw1, nb1):
    out_specs = [pl.BlockSpec((T, SIZE), lambda i: (i, 0))] * 2
    out_shape = [jax.ShapeDtypeStruct((NP, SIZE), jnp.float32)] * 2

    def body(*refs):
        _benc_body(combine_out, *refs[:20], refs[20:])

    return pl.pallas_call(
        body,
        grid=(GRID,),
        in_specs=[pl.BlockSpec((T, SIZE), lambda i: (i, 0)),
                  pl.BlockSpec((T, SIZE), lambda i: (i, 0)),
                  pl.BlockSpec((E, SIZE), lambda i: (i, 0)),
                  pl.BlockSpec((E, 32), lambda i: (i, 0)),
                  _wspec((SIZE, HA)), _wspec((1, HA)),
                  _wspec((SIZE, HA)), _wspec((32, HA)), _wspec((1, HA)),
                  _wspec((SIZE, HA)), _wspec((32, HA)), _wspec((1, HA)),
                  _wspec((HA, SIZE)), _wspec((1, SIZE)),
                  _wspec((SIZE, SIZE)), _wspec((1, SIZE)),
                  _wspec((SIZE, SIZE)), _wspec((1, SIZE)),
                  _wspec((HA, HEADS)), _wspec((HEADS, HA))],
        out_specs=out_specs,
        out_shape=out_shape,
    )(feats, loc, lg, relp, wq, bq, wkf, wkr, bk, wvf, wvr, bv, wo, bo,
      nw0, nb0, nw1, nb1, jnp.asarray(_SEG), jnp.asarray(_SEGT))


def _bdec_body(final, feats_ref, loc_ref, lgd_ref, enc_ref, rel_ref,
               wq, bq, wkf, wkr, wks, bk, wvf, wvr, wvs, bv, wo, bo,
               nw0, nb0, nw1, nb1, seg, segt, out_ref=None):
    f32 = jnp.float32
    i = pl.program_id(0)
    rel = rel_ref[...]
    eidx = lax.broadcasted_iota(jnp.int32, (E, 1), 0)
    nid = (i * T + eidx // K).astype(f32)
    pre = rel[:, 26:27] < nid
    lgd = lgd_ref[...]
    encg = enc_ref[...]
    feat_part = jnp.where(pre, lgd, encg)
    seqg = rel[:, 25:26].astype(jnp.int32)
    io32 = lax.broadcasted_iota(jnp.int32, (E, 32), 1)
    oh = jnp.where((seqg == io32) & pre, 1.0, 0.0)
    loc = loc_ref[...]
    q = jnp.dot(loc, wq[...], preferred_element_type=f32) + bq[...]
    k = (jnp.dot(feat_part, wkf[...], preferred_element_type=f32)
         + jnp.dot(rel, wkr[...], preferred_element_type=f32)
         + jnp.dot(oh, wks[...], preferred_element_type=f32) + bk[...])
    v = (jnp.dot(feat_part, wvf[...], preferred_element_type=f32)
         + jnp.dot(rel, wvr[...], preferred_element_type=f32)
         + jnp.dot(oh, wvs[...], preferred_element_type=f32) + bv[...])
    o = _attention_core(q, k, v, seg, segt)
    att = jnp.dot(o, wo[...], preferred_element_type=f32) + bo[...]
    out = feats_ref[...] + att
    if final:
        wpost, bpost = nw0, nb0
        out_ref[...] = jnp.dot(out, wpost[...], preferred_element_type=f32) + bpost[...]
    else:
        nloc = _next_local(out, nw0, nb0, nw1, nb1)
        out_ref[0][...] = out
        out_ref[1][...] = nloc


def _bdec_call(final, feats, loc, lgd, encg, relp,
               wq, bq, wkf, wkr, wks, bk, wvf, wvr, wvs, bv, wo, bo,
               nw0, nb0, nw1, nb1, feats_spec, loc_spec, lgd_spec, enc_spec):
    if final:
        out_specs = pl.BlockSpec((T, 32), lambda i: (i, 0))
        out_shape = jax.ShapeDtypeStruct((NP, 32), jnp.float32)
        mlp_specs = [_wspec((SIZE, 32)), _wspec((1, 32))]
        mlp_args = (nw0, nb0)
    else:
        out_specs = [pl.BlockSpec((T, SIZE), lambda i: (i, 0))] * 2
        out_shape = [jax.ShapeDtypeStruct((NP, SIZE), jnp.float32)] * 2
        mlp_specs = [_wspec((SIZE, SIZE)), _wspec((1, SIZE)),
                     _wspec((SIZE, SIZE)), _wspec((1, SIZE))]
        mlp_args = (nw0, nb0, nw1, nb1)

    def body(*refs):
        if final:
            ins = list(refs[:19]) + [None, None] + list(refs[19:21])
            _bdec_body(final, *ins, refs[21])
        else:
            _bdec_body(final, *refs[:23], refs[23:])

    return pl.pallas_call(
        body,
        grid=(GRID,),
        in_specs=[feats_spec, loc_spec, lgd_spec, enc_spec,
                  pl.BlockSpec((E, 32), lambda i: (i, 0)),
                  _wspec((SIZE, HA)), _wspec((1, HA)),
                  _wspec((SIZE, HA)), _wspec((32, HA)), _wspec((32, HA)),
                  _wspec((1, HA)),
                  _wspec((SIZE, HA)), _wspec((32, HA)), _wspec((32, HA)),
                  _wspec((1, HA)),
                  _wspec((HA, SIZE)), _wspec((1, SIZE))] + mlp_specs
                 + [_wspec((HA, HEADS)), _wspec((HEADS, HA))],
        out_specs=out_specs,
        out_shape=out_shape,
    )(feats, loc, lgd, encg, relp, wq, bq, wkf, wkr, wks, bk,
      wvf, wvr, wvs, bv, wo, bo, *mlp_args,
      jnp.asarray(_SEG), jnp.asarray(_SEGT))


def _row(b):
    return b.reshape(1, -1)


def _pad_rows(w, rows):
    return jnp.pad(w, ((0, rows - w.shape[0]), (0, 0)))


def _enc_weights(bp):
    wk = bp["Wk"]["W"]
    wv = bp["Wv"]["W"]
    return dict(
        wq=bp["Wq"]["W"], bq=_row(bp["Wq"]["b"]),
        wkf=wk[:SIZE], wkr=_pad_rows(wk[SIZE:SIZE + 25], 32),
        bk=_row(bp["Wk"]["b"]),
        wvf=wv[:SIZE], wvr=_pad_rows(wv[SIZE:SIZE + 25], 32),
        bv=_row(bp["Wv"]["b"]),
        wo=bp["Wo"]["W"], bo=_row(bp["Wo"]["b"]),
    )


def _dec_weights(bp):
    wk = bp["Wk"]["W"]
    wv = bp["Wv"]["W"]
    d = _enc_weights(bp)
    d["wks"] = _pad_rows(wk[SIZE + 25:SIZE + 25 + SEQ_SIZE], 32)
    d["wvs"] = _pad_rows(wv[SIZE + 25:SIZE + 25 + SEQ_SIZE], 32)
    return d


def _mlp_weights(bp):
    return (bp["mlp0"]["W"], _row(bp["mlp0"]["b"]),
            bp["mlp1"]["W"], _row(bp["mlp1"]["b"]))


def kernel(features, sequence, distances, structure, params):
    f32 = jnp.float32
    fp = jnp.pad(features.astype(f32), ((0, NP - N), (0, 0)))
    dist16 = jnp.concatenate(
        [distances.astype(f32), sequence.astype(f32)[:, None],
         jnp.zeros((N, 2), f32)], axis=1)
    dist16p = jnp.pad(dist16, ((0, NP - N), (0, 0)))
    connp = jnp.pad(structure.astype(jnp.int32), ((0, NP - N), (0, 0)))
    idx2d = connp.reshape(EP // CH, CH)

    eb0, eb1 = params["enc_blocks"]
    db0, db1 = params["dec_blocks"]
    ew0, ew1 = _enc_weights(eb0), _enc_weights(eb1)
    dw0, dw1 = _dec_weights(db0), _dec_weights(db1)
    wpost = _pad_rows(params["post"]["W"].T, 32).T
    bpost = _pad_rows(_row(params["post"]["b"]).T, 32).T

    out0, loc1, dist16f = _a1_call(fp, dist16p, params["enc_pre"]["W"],
                                   _row(params["enc_pre"]["b"]),
                                   *_mlp_weights(eb0))

    distg = _sc_gather(dist16f, idx2d, SIZE)
    relp = _rel_call(dist16f, distg)

    lg1 = _sc_gather(loc1, idx2d, SIZE)
    out1, loc2 = _benc_call(False, out0, loc1, lg1, relp,
                            **{k: v for k, v in ew0.items()},
                            nw0=_mlp_weights(eb1)[0], nb0=_mlp_weights(eb1)[1],
                            nw1=_mlp_weights(eb1)[2], nb1=_mlp_weights(eb1)[3])

    lg2 = _sc_gather(loc2, idx2d, SIZE)
    out2, loc3 = _benc_call(False, out1, loc2, lg2, relp,
                            **{k: v for k, v in ew1.items()},
                            nw0=_mlp_weights(db0)[0], nb0=_mlp_weights(db0)[1],
                            nw1=_mlp_weights(db0)[2], nb1=_mlp_weights(db0)[3])

    lgE = _sc_gather(out2, idx2d, SIZE)
    lg3 = _sc_gather(loc3, idx2d, SIZE)

    tspec0 = pl.BlockSpec((T, SIZE), lambda i: (i, 0))
    espec0 = pl.BlockSpec((E, SIZE), lambda i: (i, 0))

    out3, loc4 = _bdec_call(False, out2, loc3, lg3, lgE, relp,
                            dw0["wq"], dw0["bq"], dw0["wkf"], dw0["wkr"],
                            dw0["wks"], dw0["bk"], dw0["wvf"], dw0["wvr"],
                            dw0["wvs"], dw0["bv"], dw0["wo"], dw0["bo"],
                            _mlp_weights(db1)[0], _mlp_weights(db1)[1],
                            _mlp_weights(db1)[2], _mlp_weights(db1)[3],
                            tspec0, tspec0, espec0, espec0)

    lg4 = _sc_gather(loc4, idx2d, SIZE)
    final32 = _bdec_call(True, out3, loc4, lg4, lgE, relp,
                         dw1["wq"], dw1["bq"], dw1["wkf"], dw1["wkr"],
                         dw1["wks"], dw1["bk"], dw1["wvf"], dw1["wvr"],
                         dw1["wvs"], dw1["bv"], dw1["wo"], dw1["bo"],
                         wpost, bpost, wpost, bpost,
                         tspec0, tspec0, espec0, espec0)

    return final32[:N, :20]
